# baseline (speedup 1.0000x reference)
import functools

import jax
import jax.numpy as jnp
from jax.experimental import pallas as pl
from jax.experimental.pallas import tpu as pltpu

_GAMMA = 0.1
_BM = 200


def _prologue_kernel(x_ref, w_ref, sc_ref, b_ref, dkw_ref, db_ref,
                     xw_ref, s_ref, d_ref):
    x = x_ref[...]
    xw_ref[...] = jnp.dot(x, w_ref[...], preferred_element_type=jnp.float32)
    s_ref[...] = jax.nn.sigmoid(
        jnp.dot(x, sc_ref[...], preferred_element_type=jnp.float32)
        + b_ref[0, 0])
    d_ref[...] = (jnp.dot(x, dkw_ref[...], preferred_element_type=jnp.float32)
                  + db_ref[0, 0])


def _prologue(x, w, scores, bias, dkw, dbias):
    n = x.shape[0]
    h = w.shape[1]
    return pl.pallas_call(
        _prologue_kernel,
        out_shape=(
            jax.ShapeDtypeStruct((n, h), jnp.float32),
            jax.ShapeDtypeStruct((n, 1), jnp.float32),
            jax.ShapeDtypeStruct((n, 1), jnp.float32),
        ),
    )(x, w, scores, bias.reshape(1, 1), dkw, dbias.reshape(1, 1))


def _layer_kernel(adj_ref, adjk_ref, xw_ref, xwb_ref, s_ref, d_ref, out_ref):
    s = s_ref[...]
    p = jnp.dot(adj_ref[...], xw_ref[...], preferred_element_type=jnp.float32)
    q = jnp.dot(adjk_ref[...], xw_ref[...], preferred_element_type=jnp.float32)
    out_ref[...] = s * p + (1.0 - s) * q + _GAMMA * d_ref[...] * xwb_ref[...]


def _layer(adj, adjk, xw, s, d):
    n = adj.shape[0]
    h = xw.shape[1]
    nm = n // _BM
    return pl.pallas_call(
        _layer_kernel,
        grid=(nm,),
        in_specs=[
            pl.BlockSpec((_BM, n), lambda i: (i, 0)),
            pl.BlockSpec((_BM, n), lambda i: (i, 0)),
            pl.BlockSpec((n, h), lambda i: (0, 0)),
            pl.BlockSpec((_BM, h), lambda i: (i, 0)),
            pl.BlockSpec((_BM, 1), lambda i: (i, 0)),
            pl.BlockSpec((_BM, 1), lambda i: (i, 0)),
        ],
        out_specs=pl.BlockSpec((_BM, h), lambda i: (i, 0)),
        out_shape=jax.ShapeDtypeStruct((n, h), jnp.float32),
        compiler_params=pltpu.CompilerParams(
            dimension_semantics=("arbitrary",)),
    )(adj, adjk, xw, xw, s, d)


def kernel(x, adj, adj_knn, W1, W2, scores0, bias0, Dk0, Dbias0,
           scores1, bias1, Dk1, Dbias1):
    xw, s, d = _prologue(x, W1, scores0, bias0, Dk0, Dbias0)
    x1 = _layer(adj, adj_knn, xw, s, d)
    xw2, s1, d1 = _prologue(x1, W2, scores1, bias1, Dk1, Dbias1)
    return _layer(adj, adj_knn, xw2, s1, d1)
